# baseline (speedup 1.0000x reference)
import jax
import jax.numpy as jnp
from jax.experimental import pallas as pl
from jax.experimental.pallas import tpu as pltpu

T, B, C, H, W, E, TOPK = 4, 16, 128, 16, 16, 8, 2
HW = H * W
NR = B * HW
N = T * NR
EPS = 1e-5


def _router_weights(xs, wr, gr, btr):
    f32 = jnp.float32
    one = jnp.float32(1.0)
    zero = jnp.float32(0.0)
    v = jnp.zeros((NR, C), f32)
    ssum = jnp.zeros((1, E), f32)
    ssq = jnp.zeros((1, E), f32)
    macc = jnp.zeros((B, E), f32)
    for t in range(T):
        v = v + (xs[t] - v) * 0.5
        mask = v >= 1.0
        sp = jnp.where(mask, one, zero)
        v = jnp.where(mask, zero, v)
        r = jnp.dot(sp, wr, preferred_element_type=f32)
        ssum = ssum + jnp.sum(r, axis=0, keepdims=True)
        ssq = ssq + jnp.sum(r * r, axis=0, keepdims=True)
        macc = macc + jnp.sum(r.reshape(B, HW, E), axis=1)
    mu = ssum / N
    var = ssq / N - mu * mu
    logits = (macc / (T * HW) - mu) * jax.lax.rsqrt(var + EPS) * gr + btr

    lmax = jnp.max(logits, axis=1, keepdims=True)
    ex = jnp.exp(logits - lmax)
    p = ex / jnp.sum(ex, axis=1, keepdims=True)
    ii = jax.lax.broadcasted_iota(jnp.int32, (B, E), 1)
    p1 = jnp.max(p, axis=1, keepdims=True)
    i1 = jnp.min(jnp.where(p == p1, ii, E), axis=1, keepdims=True)
    pm = jnp.where(ii == i1, -jnp.inf, p)
    p2 = jnp.max(pm, axis=1, keepdims=True)
    i2 = jnp.min(jnp.where(pm == p2, ii, E), axis=1, keepdims=True)
    keep = (ii == i1) | (ii == i2)
    return jnp.where(keep, p, 0.0) / (p1 + p2)


def _expert(xs, acc, shift, wcol, w1e, w2e, g1e, bt1e, g2e, bt2e, inv_tau):
    f32 = jnp.float32
    one = jnp.float32(1.0)
    zero = jnp.float32(0.0)

    v = jnp.zeros((NR, C), f32)
    h1 = []
    s1 = jnp.zeros((1, C), f32)
    q1 = jnp.zeros((1, C), f32)
    for t in range(T):
        v = v + (xs[t] - v) * inv_tau
        mask = v >= 1.0
        sp = jnp.where(mask, one, zero)
        v = jnp.where(mask, zero, v)
        h = jnp.dot(sp, w1e, preferred_element_type=f32)
        s1 = s1 + jnp.sum(h, axis=0, keepdims=True)
        q1 = q1 + jnp.sum(h * h, axis=0, keepdims=True)
        h1.append(h)
    mean1 = s1 / N
    sc1 = g1e * jax.lax.rsqrt(q1 / N - mean1 * mean1 + EPS)
    sh1 = bt1e - mean1 * sc1

    v = jnp.zeros((NR, C), f32)
    h2 = []
    s2 = jnp.zeros((1, C), f32)
    q2 = jnp.zeros((1, C), f32)
    for t in range(T):
        hAt = xs[t] + h1[t] * sc1 + sh1
        v = v + (hAt - v) * inv_tau
        mask = v >= 1.0
        sp = jnp.where(mask, one, zero)
        v = jnp.where(mask, zero, v)
        h = jnp.dot(sp, w2e, preferred_element_type=f32)
        s2 = s2 + jnp.sum(h, axis=0, keepdims=True)
        q2 = q2 + jnp.sum(h * h, axis=0, keepdims=True)
        h2.append(h)
    mean2 = s2 / N
    sc2 = g2e * jax.lax.rsqrt(q2 / N - mean2 * mean2 + EPS)
    sh2 = bt2e - mean2 * sc2

    shift = shift + wcol * (sh1 + sh2)
    ws1 = wcol.reshape(B, 1, 1) * sc1.reshape(1, 1, C)
    ws2 = wcol.reshape(B, 1, 1) * sc2.reshape(1, 1, C)
    for t in range(T):
        a3 = acc[t].reshape(B, HW, C) \
            + h1[t].reshape(B, HW, C) * ws1 \
            + h2[t].reshape(B, HW, C) * ws2
        acc[t] = a3.reshape(NR, C)
    return acc, shift


def _fused_full(x_ref, w1_ref, g1_ref, bt1_ref, w2_ref, g2_ref, bt2_ref,
                wr_ref, gr_ref, btr_ref, taus_ref, o_ref):
    f32 = jnp.float32
    xs = [x_ref[t] for t in range(T)]
    wdense = _router_weights(xs, wr_ref[...], gr_ref[...], btr_ref[...])
    wsum = jnp.sum(wdense, axis=1, keepdims=True)

    acc = [jnp.zeros((NR, C), f32) for _ in range(T)]
    shift = jnp.zeros((B, C), f32)
    for e in range(E):
        acc, shift = _expert(
            xs, acc, shift, wdense[:, e:e + 1], w1_ref[e], w2_ref[e],
            g1_ref[e:e + 1], bt1_ref[e:e + 1], g2_ref[e:e + 1],
            bt2_ref[e:e + 1], 1.0 / taus_ref[0, e])

    swb = wsum.reshape(B, 1, 1)
    shb = shift.reshape(B, 1, C)
    for t in range(T):
        o3 = xs[t].reshape(B, HW, C) * swb + acc[t].reshape(B, HW, C) + shb
        o_ref[t] = o3.reshape(NR, C)


_CPARAMS = pltpu.CompilerParams(vmem_limit_bytes=128 * 1024 * 1024)


def _run_single(xt, W1t, g1, bt1, W2t, g2, bt2, Wrt, gr2, btr2, taus2):
    return pl.pallas_call(
        _fused_full,
        out_shape=jax.ShapeDtypeStruct((T, NR, C), jnp.float32),
        compiler_params=_CPARAMS,
    )(xt, W1t, g1, bt1, W2t, g2, bt2, Wrt, gr2, btr2, taus2)


def kernel(x, W1, b1, g1, bt1, W2, b2, g2, bt2, Wr, brv, gr, btr, taus):
    xt = x.transpose(0, 1, 3, 4, 2).reshape(T, NR, C)
    args = (xt, W1.transpose(0, 2, 1), g1, bt1, W2.transpose(0, 2, 1),
            g2, bt2, Wr.T, gr.reshape(1, E), btr.reshape(1, E),
            taus.reshape(1, E))
    out = _run_single(*args)
    return out.reshape(T, B, H, W, C).transpose(0, 1, 4, 2, 3)


# speedup vs baseline: 1.0051x; 1.0051x over previous
import jax
import jax.numpy as jnp
from jax.experimental import pallas as pl
from jax.experimental.pallas import tpu as pltpu

T, B, C, H, W, E, TOPK = 4, 16, 128, 16, 16, 8, 2
HW = H * W
NR = B * HW
N = T * NR
EPS = 1e-5


def _router_weights(xs, wr, gr, btr):
    f32 = jnp.float32
    one = jnp.float32(1.0)
    zero = jnp.float32(0.0)
    v = jnp.zeros((NR, C), f32)
    ssum = jnp.zeros((1, E), f32)
    ssq = jnp.zeros((1, E), f32)
    macc = jnp.zeros((B, E), f32)
    for t in range(T):
        v = v + (xs[t] - v) * 0.5
        mask = v >= 1.0
        sp = jnp.where(mask, one, zero)
        v = jnp.where(mask, zero, v)
        r = jnp.dot(sp, wr, preferred_element_type=f32)
        ssum = ssum + jnp.sum(r, axis=0, keepdims=True)
        ssq = ssq + jnp.sum(r * r, axis=0, keepdims=True)
        macc = macc + jnp.sum(r.reshape(B, HW, E), axis=1)
    mu = ssum / N
    var = ssq / N - mu * mu
    logits = (macc / (T * HW) - mu) * jax.lax.rsqrt(var + EPS) * gr + btr

    lmax = jnp.max(logits, axis=1, keepdims=True)
    ex = jnp.exp(logits - lmax)
    p = ex / jnp.sum(ex, axis=1, keepdims=True)
    ii = jax.lax.broadcasted_iota(jnp.int32, (B, E), 1)
    p1 = jnp.max(p, axis=1, keepdims=True)
    i1 = jnp.min(jnp.where(p == p1, ii, E), axis=1, keepdims=True)
    pm = jnp.where(ii == i1, -jnp.inf, p)
    p2 = jnp.max(pm, axis=1, keepdims=True)
    i2 = jnp.min(jnp.where(pm == p2, ii, E), axis=1, keepdims=True)
    keep = (ii == i1) | (ii == i2)
    return jnp.where(keep, p, 0.0) / (p1 + p2)


def _expert(xs, acc, shift, wcol, w1e, w2e, g1e, bt1e, g2e, bt2e, inv_tau):
    f32 = jnp.float32
    one = jnp.float32(1.0)
    zero = jnp.float32(0.0)

    v = jnp.zeros((NR, C), f32)
    h1 = []
    s1 = jnp.zeros((1, C), f32)
    q1 = jnp.zeros((1, C), f32)
    for t in range(T):
        v = v + (xs[t] - v) * inv_tau
        mask = v >= 1.0
        sp = jnp.where(mask, one, zero)
        v = jnp.where(mask, zero, v)
        h = jnp.dot(sp, w1e, preferred_element_type=f32)
        s1 = s1 + jnp.sum(h, axis=0, keepdims=True)
        q1 = q1 + jnp.sum(h * h, axis=0, keepdims=True)
        h1.append(h)
    mean1 = s1 / N
    sc1 = g1e * jax.lax.rsqrt(q1 / N - mean1 * mean1 + EPS)
    sh1 = bt1e - mean1 * sc1
    hA = [xs[t] + h1[t] * sc1 + sh1 for t in range(T)]

    v = jnp.zeros((NR, C), f32)
    h2 = []
    s2 = jnp.zeros((1, C), f32)
    q2 = jnp.zeros((1, C), f32)
    for t in range(T):
        v = v + (hA[t] - v) * inv_tau
        mask = v >= 1.0
        sp = jnp.where(mask, one, zero)
        v = jnp.where(mask, zero, v)
        h = jnp.dot(sp, w2e, preferred_element_type=f32)
        s2 = s2 + jnp.sum(h, axis=0, keepdims=True)
        q2 = q2 + jnp.sum(h * h, axis=0, keepdims=True)
        h2.append(h)
    mean2 = s2 / N
    sc2 = g2e * jax.lax.rsqrt(q2 / N - mean2 * mean2 + EPS)
    sh2 = bt2e - mean2 * sc2

    shift = shift + wcol * (sh1 + sh2)
    ws1 = wcol.reshape(B, 1, 1) * sc1.reshape(1, 1, C)
    ws2 = wcol.reshape(B, 1, 1) * sc2.reshape(1, 1, C)
    for t in range(T):
        a3 = acc[t].reshape(B, HW, C) \
            + h1[t].reshape(B, HW, C) * ws1 \
            + h2[t].reshape(B, HW, C) * ws2
        acc[t] = a3.reshape(NR, C)
    return acc, shift


def _fused_full(x_ref, w1_ref, g1_ref, bt1_ref, w2_ref, g2_ref, bt2_ref,
                wr_ref, gr_ref, btr_ref, taus_ref, o_ref):
    f32 = jnp.float32
    xs = [x_ref[t] for t in range(T)]
    wdense = _router_weights(xs, wr_ref[...], gr_ref[...], btr_ref[...])
    wsum = jnp.sum(wdense, axis=1, keepdims=True)

    acc = [jnp.zeros((NR, C), f32) for _ in range(T)]
    shift = jnp.zeros((B, C), f32)
    for e in range(E):
        acc, shift = _expert(
            xs, acc, shift, wdense[:, e:e + 1], w1_ref[e], w2_ref[e],
            g1_ref[e:e + 1], bt1_ref[e:e + 1], g2_ref[e:e + 1],
            bt2_ref[e:e + 1], 1.0 / taus_ref[0, e])

    swb = wsum.reshape(B, 1, 1)
    shb = shift.reshape(B, 1, C)
    for t in range(T):
        o3 = xs[t].reshape(B, HW, C) * swb + acc[t].reshape(B, HW, C) + shb
        o_ref[t] = o3.reshape(NR, C)


_CPARAMS = pltpu.CompilerParams(vmem_limit_bytes=128 * 1024 * 1024)


def _run_single(xt, W1t, g1, bt1, W2t, g2, bt2, Wrt, gr2, btr2, taus2):
    return pl.pallas_call(
        _fused_full,
        out_shape=jax.ShapeDtypeStruct((T, NR, C), jnp.float32),
        compiler_params=_CPARAMS,
    )(xt, W1t, g1, bt1, W2t, g2, bt2, Wrt, gr2, btr2, taus2)


def kernel(x, W1, b1, g1, bt1, W2, b2, g2, bt2, Wr, brv, gr, btr, taus):
    xt = x.transpose(0, 1, 3, 4, 2).reshape(T, NR, C)
    args = (xt, W1.transpose(0, 2, 1), g1, bt1, W2.transpose(0, 2, 1),
            g2, bt2, Wr.T, gr.reshape(1, E), btr.reshape(1, E),
            taus.reshape(1, E))
    out = _run_single(*args)
    return out.reshape(T, B, H, W, C).transpose(0, 1, 4, 2, 3)
